# Initial kernel scaffold; baseline (speedup 1.0000x reference)
import jax
import jax.numpy as jnp
from jax.experimental import pallas as pl


def kernel(x, edge_index, params):
    raise NotImplementedError("write your pallas kernel here")


# speedup vs baseline: 3.0988x; 3.0988x over previous
import functools

import jax
import jax.numpy as jnp
from jax import lax
from jax.experimental import pallas as pl
from jax.experimental.pallas import tpu as pltpu
from jax.experimental.pallas import tpu_sc as plsc

N = 10000
E = 320000
D = 128

NC = 2
NS = 16
NW = NC * NS
EPW = E // NW
CH = 80
NCH = EPW // CH
ZCH = 400
NZC = N // ZCH

@functools.lru_cache(maxsize=1)
def _sc_kernels():
    mesh = plsc.VectorSubcoreMesh(core_axis_name="c", subcore_axis_name="s",
                                  num_cores=NC, num_subcores=NS)

    @functools.partial(
        pl.kernel,
        mesh=mesh,
        out_type=(jax.ShapeDtypeStruct((E, D), jnp.float32),
                  jax.ShapeDtypeStruct((E, D), jnp.float32)),
        scratch_types=[
            pltpu.VMEM((CH,), jnp.int32),
            pltpu.VMEM((CH,), jnp.int32),
            pltpu.VMEM((CH, D), jnp.float32),
            pltpu.VMEM((CH, D), jnp.float32),
            pltpu.SemaphoreType.DMA,
            pltpu.SemaphoreType.DMA,
        ],
    )
    def sc_gather(a_hbm, b_hbm, src_hbm, dst_hbm, outa_hbm, outb_hbm,
                  sidx, didx, rowsa, rowsb, sema, semb):
        wid = lax.axis_index("s") * NC + lax.axis_index("c")
        base = wid * EPW

        def body(i, carry):
            off = base + i * CH
            pltpu.sync_copy(src_hbm.at[pl.ds(off, CH)], sidx)
            pltpu.sync_copy(dst_hbm.at[pl.ds(off, CH)], didx)
            ca = pltpu.async_copy(a_hbm.at[sidx], rowsa, sema)
            cb = pltpu.async_copy(b_hbm.at[didx], rowsb, semb)
            ca.wait()
            cb.wait()
            pltpu.sync_copy(rowsa, outa_hbm.at[pl.ds(off, CH)])
            pltpu.sync_copy(rowsb, outb_hbm.at[pl.ds(off, CH)])
            return carry

        lax.fori_loop(0, NCH, body, 0)

    @functools.partial(
        pl.kernel,
        mesh=mesh,
        out_type=(jax.ShapeDtypeStruct((N, D), jnp.float32),
                  jax.ShapeDtypeStruct((N, D), jnp.float32)),
        scratch_types=[
            pltpu.VMEM((CH,), jnp.int32),
            pltpu.VMEM((CH, D), jnp.float32),
            pltpu.VMEM_SHARED((N, D), jnp.float32),
        ],
    )
    def sc_scatter(m_hbm, dst_hbm, zeros_hbm, out0_hbm, out1_hbm,
                   didx, rows, acc):
        cid = lax.axis_index("c")
        sid = lax.axis_index("s")
        wid = sid * NC + cid

        def chunk_copy(ci, src_ref, dst_ref):
            sl = pl.ds(ci * ZCH, ZCH)
            pltpu.sync_copy(src_ref.at[sl], dst_ref.at[sl])

        chunk_copy(sid, zeros_hbm, acc)

        @pl.when(sid + NS < NZC)
        def _():
            chunk_copy(sid + NS, zeros_hbm, acc)

        plsc.subcore_barrier()

        base = wid * EPW

        def body(i, carry):
            off = base + i * CH
            pltpu.sync_copy(dst_hbm.at[pl.ds(off, CH)], didx)
            pltpu.sync_copy(m_hbm.at[pl.ds(off, CH)], rows)
            pltpu.sync_copy(rows, acc.at[didx], add=True)
            return carry

        lax.fori_loop(0, NCH, body, 0)
        plsc.subcore_barrier()

        @pl.when(cid == 0)
        def _():
            chunk_copy(sid, acc, out0_hbm)

            @pl.when(sid + NS < NZC)
            def _():
                chunk_copy(sid + NS, acc, out0_hbm)

        @pl.when(cid == 1)
        def _():
            chunk_copy(sid, acc, out1_hbm)

            @pl.when(sid + NS < NZC)
            def _():
                chunk_copy(sid + NS, acc, out1_hbm)

    return sc_gather, sc_scatter


BN = 2000
BE = 4000


def _pre_body(h_ref, wst_ref, wdt_ref, b1_ref, a_ref, b_ref):
    h = h_ref[...]
    a_ref[...] = jnp.dot(h, wst_ref[...], preferred_element_type=jnp.float32)
    b_ref[...] = (jnp.dot(h, wdt_ref[...], preferred_element_type=jnp.float32)
                  + b1_ref[...])


_tc_pre = pl.pallas_call(
    _pre_body,
    grid=(N // BN,),
    in_specs=[
        pl.BlockSpec((BN, D), lambda i: (i, 0)),
        pl.BlockSpec((D, D), lambda i: (0, 0)),
        pl.BlockSpec((D, D), lambda i: (0, 0)),
        pl.BlockSpec((1, D), lambda i: (0, 0)),
    ],
    out_specs=(pl.BlockSpec((BN, D), lambda i: (i, 0)),
               pl.BlockSpec((BN, D), lambda i: (i, 0))),
    out_shape=(jax.ShapeDtypeStruct((N, D), jnp.float32),
               jax.ShapeDtypeStruct((N, D), jnp.float32)),
)


def _edge_body(pa_ref, pb_ref, w2t_ref, b2_ref, m_ref):
    z = jnp.maximum(pa_ref[...] + pb_ref[...], 0.0)
    m_ref[...] = jnp.tanh(
        jnp.dot(z, w2t_ref[...], preferred_element_type=jnp.float32)
        + b2_ref[...])


_tc_edge = pl.pallas_call(
    _edge_body,
    grid=(E // BE,),
    in_specs=[
        pl.BlockSpec((BE, D), lambda i: (i, 0)),
        pl.BlockSpec((BE, D), lambda i: (i, 0)),
        pl.BlockSpec((D, D), lambda i: (0, 0)),
        pl.BlockSpec((1, D), lambda i: (0, 0)),
    ],
    out_specs=pl.BlockSpec((BE, D), lambda i: (i, 0)),
    out_shape=jax.ShapeDtypeStruct((E, D), jnp.float32),
)


def _node_body(g0_ref, g1_ref, h_ref, wa_ref, ba_ref, wb_ref, bb_ref,
               g_ref, beta_ref, out_ref, *, ln):
    agg = g0_ref[...] + g1_ref[...]
    cat = jnp.concatenate([agg, h_ref[...]], axis=1)
    t = jnp.maximum(
        jnp.dot(cat, wa_ref[...], preferred_element_type=jnp.float32)
        + ba_ref[...], 0.0)
    y = jnp.tanh(
        jnp.dot(t, wb_ref[...], preferred_element_type=jnp.float32)
        + bb_ref[...])
    if ln:
        r = jnp.maximum(y, 0.0)
        mu = jnp.mean(r, axis=1, keepdims=True)
        var = jnp.mean((r - mu) ** 2, axis=1, keepdims=True)
        y = (r - mu) * jax.lax.rsqrt(var + 1e-5) * g_ref[...] + beta_ref[...]
    out_ref[...] = y


def _make_tc_node(ln):
    return pl.pallas_call(
        functools.partial(_node_body, ln=ln),
        grid=(N // BN,),
        in_specs=[
            pl.BlockSpec((BN, D), lambda i: (i, 0)),
            pl.BlockSpec((BN, D), lambda i: (i, 0)),
            pl.BlockSpec((BN, D), lambda i: (i, 0)),
            pl.BlockSpec((2 * D, 2 * D), lambda i: (0, 0)),
            pl.BlockSpec((1, 2 * D), lambda i: (0, 0)),
            pl.BlockSpec((2 * D, D), lambda i: (0, 0)),
            pl.BlockSpec((1, D), lambda i: (0, 0)),
            pl.BlockSpec((1, D), lambda i: (0, 0)),
            pl.BlockSpec((1, D), lambda i: (0, 0)),
        ],
        out_specs=pl.BlockSpec((BN, D), lambda i: (i, 0)),
        out_shape=jax.ShapeDtypeStruct((N, D), jnp.float32),
    )


_tc_node_ln = _make_tc_node(True)
_tc_node_plain = _make_tc_node(False)


def _node_weights(p):
    w1a, b1a = p['n1a']
    w2a, b2a = p['n2a']
    w1b, b1b = p['n1b']
    w2b, b2b = p['n2b']
    wa = jnp.zeros((2 * D, 2 * D), jnp.float32)
    wa = wa.at[:D, :D].set(w1a.T).at[D:, D:].set(w2a.T)
    ba = jnp.concatenate([b1a, b2a])[None]
    wb = jnp.zeros((2 * D, D), jnp.float32)
    wb = wb.at[:D, :D // 2].set(w1b.T).at[D:, D // 2:].set(w2b.T)
    bb = jnp.concatenate([b1b, b2b])[None]
    return wa, ba, wb, bb


def kernel(x, edge_index, params):
    sc_gather, sc_scatter = _sc_kernels()
    src = edge_index[0]
    dst = edge_index[1]
    zeros_tbl = jnp.zeros((N, D), jnp.float32)

    def layer(h, p):
        w1, b1 = p['e1']
        a, b = _tc_pre(h, w1[:, :D].T, w1[:, D:].T, b1[None])
        pa, pb = sc_gather(a, b, src, dst)
        m = _tc_edge(pa, pb, p['e2'][0].T, p['e2'][1][None])
        return sc_scatter(m, dst, zeros_tbl)

    g0, g1 = layer(x, params['l0'])
    wa, ba, wb, bb = _node_weights(params['l0'])
    h1 = _tc_node_ln(g0, g1, x, wa, ba, wb, bb,
                     params['ln0']['g'][None], params['ln0']['b'][None])
    g0, g1 = layer(h1, params['l1'])
    wa, ba, wb, bb = _node_weights(params['l1'])
    dummy = jnp.zeros((1, D), jnp.float32)
    return _tc_node_plain(g0, g1, h1, wa, ba, wb, bb, dummy, dummy)


# speedup vs baseline: 3.9268x; 1.2672x over previous
import functools

import jax
import jax.numpy as jnp
from jax import lax
from jax.experimental import pallas as pl
from jax.experimental.pallas import tpu as pltpu
from jax.experimental.pallas import tpu_sc as plsc

N = 10000
E = 320000
D = 128

NC = 2
NS = 16
NW = NC * NS
EPW = E // NW
CH = 96
NCH = EPW // CH
NJ = NCH // 2
TAIL = EPW - NCH * CH
ZCH = 400
NZC = N // ZCH


@functools.lru_cache(maxsize=1)
def _sc_kernels():
    mesh = plsc.VectorSubcoreMesh(core_axis_name="c", subcore_axis_name="s",
                                  num_cores=NC, num_subcores=NS)

    def add_rows(ra, rb, n_rows):
        def body(r, carry):
            for c in range(D // 16):
                sl = pl.ds(c * 16, 16)
                ra[r, sl] = ra[r, sl] + rb[r, sl]
            return carry

        lax.fori_loop(0, n_rows, body, 0)

    @functools.partial(
        pl.kernel,
        mesh=mesh,
        out_type=jax.ShapeDtypeStruct((E, D), jnp.float32),
        scratch_types=[
            pltpu.VMEM((CH,), jnp.int32),
            pltpu.VMEM((CH,), jnp.int32),
            pltpu.VMEM((CH,), jnp.int32),
            pltpu.VMEM((CH,), jnp.int32),
            pltpu.VMEM((CH, D), jnp.float32),
            pltpu.VMEM((CH, D), jnp.float32),
            pltpu.VMEM((CH, D), jnp.float32),
            pltpu.VMEM((CH, D), jnp.float32),
            pltpu.SemaphoreType.DMA,
            pltpu.SemaphoreType.DMA,
            pltpu.SemaphoreType.DMA,
            pltpu.SemaphoreType.DMA,
            pltpu.SemaphoreType.DMA,
            pltpu.SemaphoreType.DMA,
        ],
    )
    def sc_gather(a_hbm, b_hbm, src_hbm, dst_hbm, out_hbm,
                  si0, di0, si1, di1, ra0, rb0, ra1, rb1,
                  ga0, gb0, ga1, gb1, ws0, ws1):
        wid = lax.axis_index("s") * NC + lax.axis_index("c")
        base = wid * EPW

        def pair(j, carry):
            o0 = base + (2 * j) * CH
            o1 = o0 + CH
            pltpu.sync_copy(src_hbm.at[pl.ds(o0, CH)], si0)
            pltpu.sync_copy(dst_hbm.at[pl.ds(o0, CH)], di0)
            pltpu.sync_copy(src_hbm.at[pl.ds(o1, CH)], si1)
            pltpu.sync_copy(dst_hbm.at[pl.ds(o1, CH)], di1)
            ca0 = pltpu.async_copy(a_hbm.at[si0], ra0, ga0)
            cb0 = pltpu.async_copy(b_hbm.at[di0], rb0, gb0)
            ca1 = pltpu.async_copy(a_hbm.at[si1], ra1, ga1)
            cb1 = pltpu.async_copy(b_hbm.at[di1], rb1, gb1)
            ca0.wait()
            cb0.wait()
            add_rows(ra0, rb0, CH)
            w0 = pltpu.async_copy(ra0, out_hbm.at[pl.ds(o0, CH)], ws0)
            ca1.wait()
            cb1.wait()
            add_rows(ra1, rb1, CH)
            w1 = pltpu.async_copy(ra1, out_hbm.at[pl.ds(o1, CH)], ws1)
            w0.wait()
            w1.wait()
            return carry

        lax.fori_loop(0, NJ, pair, 0)

        ot = base + EPW - CH
        pltpu.sync_copy(src_hbm.at[pl.ds(ot, CH)], si0)
        pltpu.sync_copy(dst_hbm.at[pl.ds(ot, CH)], di0)
        ca = pltpu.async_copy(a_hbm.at[si0], ra0, ga0)
        cb = pltpu.async_copy(b_hbm.at[di0], rb0, gb0)
        ca.wait()
        cb.wait()
        add_rows(ra0, rb0, CH)
        pltpu.sync_copy(ra0, out_hbm.at[pl.ds(ot, CH)])

    @functools.partial(
        pl.kernel,
        mesh=mesh,
        out_type=(jax.ShapeDtypeStruct((N, D), jnp.float32),
                  jax.ShapeDtypeStruct((N, D), jnp.float32)),
        scratch_types=[
            pltpu.VMEM((CH,), jnp.int32),
            pltpu.VMEM((CH,), jnp.int32),
            pltpu.VMEM((CH, D), jnp.float32),
            pltpu.VMEM((CH, D), jnp.float32),
            pltpu.VMEM((TAIL,), jnp.int32),
            pltpu.VMEM((TAIL, D), jnp.float32),
            pltpu.VMEM_SHARED((N, D), jnp.float32),
            pltpu.SemaphoreType.DMA,
            pltpu.SemaphoreType.DMA,
        ],
    )
    def sc_scatter(m_hbm, dst_hbm, zeros_hbm, out0_hbm, out1_hbm,
                   di0, di1, r0, r1, dit, rt, acc, s0, s1):
        cid = lax.axis_index("c")
        sid = lax.axis_index("s")
        wid = sid * NC + cid
        base = wid * EPW

        def chunk_copy(ci, src_ref, dst_ref):
            sl = pl.ds(ci * ZCH, ZCH)
            pltpu.sync_copy(src_ref.at[sl], dst_ref.at[sl])

        chunk_copy(sid, zeros_hbm, acc)

        @pl.when(sid + NS < NZC)
        def _():
            chunk_copy(sid + NS, zeros_hbm, acc)

        plsc.subcore_barrier()

        def load(off, di, rows, sem):
            c1 = pltpu.async_copy(dst_hbm.at[pl.ds(off, CH)], di, sem)
            c2 = pltpu.async_copy(m_hbm.at[pl.ds(off, CH)], rows, sem)
            return c1, c2

        def pair(j, carry):
            o0 = base + (2 * j) * CH
            o1 = o0 + CH
            l0a, l0b = load(o0, di0, r0, s0)
            l1a, l1b = load(o1, di1, r1, s1)
            l0a.wait()
            l0b.wait()
            pltpu.sync_copy(r0, acc.at[di0], add=True)
            l1a.wait()
            l1b.wait()
            pltpu.sync_copy(r1, acc.at[di1], add=True)
            return carry

        lax.fori_loop(0, NJ, pair, 0)

        ot = base + NCH * CH
        pltpu.sync_copy(dst_hbm.at[pl.ds(ot, TAIL)], dit)
        pltpu.sync_copy(m_hbm.at[pl.ds(ot, TAIL)], rt)
        pltpu.sync_copy(rt, acc.at[dit], add=True)

        plsc.subcore_barrier()

        @pl.when(cid == 0)
        def _():
            chunk_copy(sid, acc, out0_hbm)

            @pl.when(sid + NS < NZC)
            def _():
                chunk_copy(sid + NS, acc, out0_hbm)

        @pl.when(cid == 1)
        def _():
            chunk_copy(sid, acc, out1_hbm)

            @pl.when(sid + NS < NZC)
            def _():
                chunk_copy(sid + NS, acc, out1_hbm)

    return sc_gather, sc_scatter


BN = 2000
BE = 4000


def _pre_body(h_ref, wst_ref, wdt_ref, b1_ref, a_ref, b_ref):
    h = h_ref[...]
    a_ref[...] = jnp.dot(h, wst_ref[...], preferred_element_type=jnp.float32)
    b_ref[...] = (jnp.dot(h, wdt_ref[...], preferred_element_type=jnp.float32)
                  + b1_ref[...])


_tc_pre = pl.pallas_call(
    _pre_body,
    grid=(N // BN,),
    in_specs=[
        pl.BlockSpec((BN, D), lambda i: (i, 0)),
        pl.BlockSpec((D, D), lambda i: (0, 0)),
        pl.BlockSpec((D, D), lambda i: (0, 0)),
        pl.BlockSpec((1, D), lambda i: (0, 0)),
    ],
    out_specs=(pl.BlockSpec((BN, D), lambda i: (i, 0)),
               pl.BlockSpec((BN, D), lambda i: (i, 0))),
    out_shape=(jax.ShapeDtypeStruct((N, D), jnp.float32),
               jax.ShapeDtypeStruct((N, D), jnp.float32)),
)


def _edge_body(pre_ref, w2t_ref, b2_ref, m_ref):
    z = jnp.maximum(pre_ref[...], 0.0)
    m_ref[...] = jnp.tanh(
        jnp.dot(z, w2t_ref[...], preferred_element_type=jnp.float32)
        + b2_ref[...])


_tc_edge = pl.pallas_call(
    _edge_body,
    grid=(E // BE,),
    in_specs=[
        pl.BlockSpec((BE, D), lambda i: (i, 0)),
        pl.BlockSpec((D, D), lambda i: (0, 0)),
        pl.BlockSpec((1, D), lambda i: (0, 0)),
    ],
    out_specs=pl.BlockSpec((BE, D), lambda i: (i, 0)),
    out_shape=jax.ShapeDtypeStruct((E, D), jnp.float32),
)


def _node_body(g0_ref, g1_ref, h_ref, wa_ref, ba_ref, wb_ref, bb_ref,
               g_ref, beta_ref, out_ref, *, ln):
    agg = g0_ref[...] + g1_ref[...]
    cat = jnp.concatenate([agg, h_ref[...]], axis=1)
    t = jnp.maximum(
        jnp.dot(cat, wa_ref[...], preferred_element_type=jnp.float32)
        + ba_ref[...], 0.0)
    y = jnp.tanh(
        jnp.dot(t, wb_ref[...], preferred_element_type=jnp.float32)
        + bb_ref[...])
    if ln:
        r = jnp.maximum(y, 0.0)
        mu = jnp.mean(r, axis=1, keepdims=True)
        var = jnp.mean((r - mu) ** 2, axis=1, keepdims=True)
        y = (r - mu) * jax.lax.rsqrt(var + 1e-5) * g_ref[...] + beta_ref[...]
    out_ref[...] = y


def _make_tc_node(ln):
    return pl.pallas_call(
        functools.partial(_node_body, ln=ln),
        grid=(N // BN,),
        in_specs=[
            pl.BlockSpec((BN, D), lambda i: (i, 0)),
            pl.BlockSpec((BN, D), lambda i: (i, 0)),
            pl.BlockSpec((BN, D), lambda i: (i, 0)),
            pl.BlockSpec((2 * D, 2 * D), lambda i: (0, 0)),
            pl.BlockSpec((1, 2 * D), lambda i: (0, 0)),
            pl.BlockSpec((2 * D, D), lambda i: (0, 0)),
            pl.BlockSpec((1, D), lambda i: (0, 0)),
            pl.BlockSpec((1, D), lambda i: (0, 0)),
            pl.BlockSpec((1, D), lambda i: (0, 0)),
        ],
        out_specs=pl.BlockSpec((BN, D), lambda i: (i, 0)),
        out_shape=jax.ShapeDtypeStruct((N, D), jnp.float32),
    )


_tc_node_ln = _make_tc_node(True)
_tc_node_plain = _make_tc_node(False)


def _node_weights(p):
    w1a, b1a = p['n1a']
    w2a, b2a = p['n2a']
    w1b, b1b = p['n1b']
    w2b, b2b = p['n2b']
    wa = jnp.zeros((2 * D, 2 * D), jnp.float32)
    wa = wa.at[:D, :D].set(w1a.T).at[D:, D:].set(w2a.T)
    ba = jnp.concatenate([b1a, b2a])[None]
    wb = jnp.zeros((2 * D, D), jnp.float32)
    wb = wb.at[:D, :D // 2].set(w1b.T).at[D:, D // 2:].set(w2b.T)
    bb = jnp.concatenate([b1b, b2b])[None]
    return wa, ba, wb, bb


def kernel(x, edge_index, params):
    sc_gather, sc_scatter = _sc_kernels()
    src = edge_index[0]
    dst = edge_index[1]
    zeros_tbl = jnp.zeros((N, D), jnp.float32)

    def layer(h, p):
        w1, b1 = p['e1']
        a, b = _tc_pre(h, w1[:, :D].T, w1[:, D:].T, b1[None])
        pre = sc_gather(a, b, src, dst)
        m = _tc_edge(pre, p['e2'][0].T, p['e2'][1][None])
        return sc_scatter(m, dst, zeros_tbl)

    g0, g1 = layer(x, params['l0'])
    wa, ba, wb, bb = _node_weights(params['l0'])
    h1 = _tc_node_ln(g0, g1, x, wa, ba, wb, bb,
                     params['ln0']['g'][None], params['ln0']['b'][None])
    g0, g1 = layer(h1, params['l1'])
    wa, ba, wb, bb = _node_weights(params['l1'])
    dummy = jnp.zeros((1, D), jnp.float32)
    return _tc_node_plain(g0, g1, h1, wa, ba, wb, bb, dummy, dummy)


# speedup vs baseline: 4.7186x; 1.2016x over previous
import functools

import jax
import jax.numpy as jnp
from jax import lax
from jax.experimental import pallas as pl
from jax.experimental.pallas import tpu as pltpu
from jax.experimental.pallas import tpu_sc as plsc

N = 10000
E = 320000
D = 128

NC = 2
NS = 16
NW = NC * NS
EPW = E // NW
CH = 128
NCH = EPW // CH
NJ = NCH // 2
TAIL = EPW - NCH * CH
ZCH = 400
NZC = N // ZCH


@functools.lru_cache(maxsize=1)
def _sc_kernels():
    mesh = plsc.VectorSubcoreMesh(core_axis_name="c", subcore_axis_name="s",
                                  num_cores=NC, num_subcores=NS)

    def add_rows(ra, rb, n_rows):
        def body(r, carry):
            for c in range(D // 16):
                sl = pl.ds(c * 16, 16)
                ra[r, sl] = ra[r, sl] + rb[r, sl]
            return carry

        lax.fori_loop(0, n_rows, body, 0)

    @functools.partial(
        pl.kernel,
        mesh=mesh,
        out_type=jax.ShapeDtypeStruct((E, D), jnp.float32),
        scratch_types=[
            pltpu.VMEM((EPW,), jnp.int32),
            pltpu.VMEM((EPW,), jnp.int32),
            pltpu.VMEM((CH, D), jnp.float32),
            pltpu.VMEM((CH, D), jnp.float32),
            pltpu.VMEM((CH, D), jnp.float32),
            pltpu.VMEM((CH, D), jnp.float32),
            pltpu.SemaphoreType.DMA,
            pltpu.SemaphoreType.DMA,
            pltpu.SemaphoreType.DMA,
            pltpu.SemaphoreType.DMA,
            pltpu.SemaphoreType.DMA,
            pltpu.SemaphoreType.DMA,
        ],
    )
    def sc_gather(a_hbm, b_hbm, src_hbm, dst_hbm, out_hbm,
                  si, di, ra0, rb0, ra1, rb1,
                  ga0, gb0, ga1, gb1, ws0, ws1):
        wid = lax.axis_index("s") * NC + lax.axis_index("c")
        base = wid * EPW
        pltpu.sync_copy(src_hbm.at[pl.ds(base, EPW)], si)
        pltpu.sync_copy(dst_hbm.at[pl.ds(base, EPW)], di)

        def pair(j, carry):
            l0 = (2 * j) * CH
            l1 = l0 + CH
            ca0 = pltpu.async_copy(a_hbm.at[si.at[pl.ds(l0, CH)]], ra0, ga0)
            cb0 = pltpu.async_copy(b_hbm.at[di.at[pl.ds(l0, CH)]], rb0, gb0)
            ca1 = pltpu.async_copy(a_hbm.at[si.at[pl.ds(l1, CH)]], ra1, ga1)
            cb1 = pltpu.async_copy(b_hbm.at[di.at[pl.ds(l1, CH)]], rb1, gb1)
            ca0.wait()
            cb0.wait()
            add_rows(ra0, rb0, CH)
            w0 = pltpu.async_copy(ra0, out_hbm.at[pl.ds(base + l0, CH)], ws0)
            ca1.wait()
            cb1.wait()
            add_rows(ra1, rb1, CH)
            w1 = pltpu.async_copy(ra1, out_hbm.at[pl.ds(base + l1, CH)], ws1)
            w0.wait()
            w1.wait()
            return carry

        lax.fori_loop(0, NJ, pair, 0)

        lt = EPW - CH
        ca = pltpu.async_copy(a_hbm.at[si.at[pl.ds(lt, CH)]], ra0, ga0)
        cb = pltpu.async_copy(b_hbm.at[di.at[pl.ds(lt, CH)]], rb0, gb0)
        ca.wait()
        cb.wait()
        add_rows(ra0, rb0, CH)
        pltpu.sync_copy(ra0, out_hbm.at[pl.ds(base + lt, CH)])

    @functools.partial(
        pl.kernel,
        mesh=mesh,
        out_type=(jax.ShapeDtypeStruct((N, D), jnp.float32),
                  jax.ShapeDtypeStruct((N, D), jnp.float32)),
        scratch_types=[
            pltpu.VMEM((EPW,), jnp.int32),
            pltpu.VMEM((CH, D), jnp.float32),
            pltpu.VMEM((CH, D), jnp.float32),
            pltpu.VMEM((TAIL, D), jnp.float32),
            pltpu.VMEM_SHARED((N, D), jnp.float32),
            pltpu.SemaphoreType.DMA,
            pltpu.SemaphoreType.DMA,
        ],
    )
    def sc_scatter(m_hbm, dst_hbm, zeros_hbm, out0_hbm, out1_hbm,
                   di, r0, r1, rt, acc, s0, s1):
        cid = lax.axis_index("c")
        sid = lax.axis_index("s")
        wid = sid * NC + cid
        base = wid * EPW

        def chunk_copy(ci, src_ref, dst_ref):
            sl = pl.ds(ci * ZCH, ZCH)
            pltpu.sync_copy(src_ref.at[sl], dst_ref.at[sl])

        chunk_copy(sid, zeros_hbm, acc)

        @pl.when(sid + NS < NZC)
        def _():
            chunk_copy(sid + NS, zeros_hbm, acc)

        pltpu.sync_copy(dst_hbm.at[pl.ds(base, EPW)], di)
        plsc.subcore_barrier()

        def pair(j, carry):
            l0 = (2 * j) * CH
            l1 = l0 + CH
            c0 = pltpu.async_copy(m_hbm.at[pl.ds(base + l0, CH)], r0, s0)
            c1 = pltpu.async_copy(m_hbm.at[pl.ds(base + l1, CH)], r1, s1)
            c0.wait()
            pltpu.sync_copy(r0, acc.at[di.at[pl.ds(l0, CH)]], add=True)
            c1.wait()
            pltpu.sync_copy(r1, acc.at[di.at[pl.ds(l1, CH)]], add=True)
            return carry

        lax.fori_loop(0, NJ, pair, 0)

        lt = NCH * CH
        pltpu.sync_copy(m_hbm.at[pl.ds(base + lt, TAIL)], rt)
        pltpu.sync_copy(rt, acc.at[di.at[pl.ds(lt, TAIL)]], add=True)

        plsc.subcore_barrier()

        @pl.when(cid == 0)
        def _():
            chunk_copy(sid, acc, out0_hbm)

            @pl.when(sid + NS < NZC)
            def _():
                chunk_copy(sid + NS, acc, out0_hbm)

        @pl.when(cid == 1)
        def _():
            chunk_copy(sid, acc, out1_hbm)

            @pl.when(sid + NS < NZC)
            def _():
                chunk_copy(sid + NS, acc, out1_hbm)

    return sc_gather, sc_scatter


BN = 2000
BE = 4000


def _pre_body(h_ref, wst_ref, wdt_ref, b1_ref, a_ref, b_ref):
    h = h_ref[...]
    a_ref[...] = jnp.dot(h, wst_ref[...], preferred_element_type=jnp.float32)
    b_ref[...] = (jnp.dot(h, wdt_ref[...], preferred_element_type=jnp.float32)
                  + b1_ref[...])


_tc_pre = pl.pallas_call(
    _pre_body,
    grid=(N // BN,),
    in_specs=[
        pl.BlockSpec((BN, D), lambda i: (i, 0)),
        pl.BlockSpec((D, D), lambda i: (0, 0)),
        pl.BlockSpec((D, D), lambda i: (0, 0)),
        pl.BlockSpec((1, D), lambda i: (0, 0)),
    ],
    out_specs=(pl.BlockSpec((BN, D), lambda i: (i, 0)),
               pl.BlockSpec((BN, D), lambda i: (i, 0))),
    out_shape=(jax.ShapeDtypeStruct((N, D), jnp.float32),
               jax.ShapeDtypeStruct((N, D), jnp.float32)),
)


def _edge_body(pre_ref, w2t_ref, b2_ref, m_ref):
    z = jnp.maximum(pre_ref[...], 0.0)
    m_ref[...] = jnp.tanh(
        jnp.dot(z, w2t_ref[...], preferred_element_type=jnp.float32)
        + b2_ref[...])


_tc_edge = pl.pallas_call(
    _edge_body,
    grid=(E // BE,),
    in_specs=[
        pl.BlockSpec((BE, D), lambda i: (i, 0)),
        pl.BlockSpec((D, D), lambda i: (0, 0)),
        pl.BlockSpec((1, D), lambda i: (0, 0)),
    ],
    out_specs=pl.BlockSpec((BE, D), lambda i: (i, 0)),
    out_shape=jax.ShapeDtypeStruct((E, D), jnp.float32),
)


def _node_body(g0_ref, g1_ref, h_ref, wa_ref, ba_ref, wb_ref, bb_ref,
               g_ref, beta_ref, out_ref, *, ln):
    agg = g0_ref[...] + g1_ref[...]
    cat = jnp.concatenate([agg, h_ref[...]], axis=1)
    t = jnp.maximum(
        jnp.dot(cat, wa_ref[...], preferred_element_type=jnp.float32)
        + ba_ref[...], 0.0)
    y = jnp.tanh(
        jnp.dot(t, wb_ref[...], preferred_element_type=jnp.float32)
        + bb_ref[...])
    if ln:
        r = jnp.maximum(y, 0.0)
        mu = jnp.mean(r, axis=1, keepdims=True)
        var = jnp.mean((r - mu) ** 2, axis=1, keepdims=True)
        y = (r - mu) * jax.lax.rsqrt(var + 1e-5) * g_ref[...] + beta_ref[...]
    out_ref[...] = y


def _make_tc_node(ln):
    return pl.pallas_call(
        functools.partial(_node_body, ln=ln),
        grid=(N // BN,),
        in_specs=[
            pl.BlockSpec((BN, D), lambda i: (i, 0)),
            pl.BlockSpec((BN, D), lambda i: (i, 0)),
            pl.BlockSpec((BN, D), lambda i: (i, 0)),
            pl.BlockSpec((2 * D, 2 * D), lambda i: (0, 0)),
            pl.BlockSpec((1, 2 * D), lambda i: (0, 0)),
            pl.BlockSpec((2 * D, D), lambda i: (0, 0)),
            pl.BlockSpec((1, D), lambda i: (0, 0)),
            pl.BlockSpec((1, D), lambda i: (0, 0)),
            pl.BlockSpec((1, D), lambda i: (0, 0)),
        ],
        out_specs=pl.BlockSpec((BN, D), lambda i: (i, 0)),
        out_shape=jax.ShapeDtypeStruct((N, D), jnp.float32),
    )


_tc_node_ln = _make_tc_node(True)
_tc_node_plain = _make_tc_node(False)


def _node_weights(p):
    w1a, b1a = p['n1a']
    w2a, b2a = p['n2a']
    w1b, b1b = p['n1b']
    w2b, b2b = p['n2b']
    wa = jnp.zeros((2 * D, 2 * D), jnp.float32)
    wa = wa.at[:D, :D].set(w1a.T).at[D:, D:].set(w2a.T)
    ba = jnp.concatenate([b1a, b2a])[None]
    wb = jnp.zeros((2 * D, D), jnp.float32)
    wb = wb.at[:D, :D // 2].set(w1b.T).at[D:, D // 2:].set(w2b.T)
    bb = jnp.concatenate([b1b, b2b])[None]
    return wa, ba, wb, bb


def kernel(x, edge_index, params):
    sc_gather, sc_scatter = _sc_kernels()
    src = edge_index[0]
    dst = edge_index[1]
    zeros_tbl = jnp.zeros((N, D), jnp.float32)

    def layer(h, p):
        w1, b1 = p['e1']
        a, b = _tc_pre(h, w1[:, :D].T, w1[:, D:].T, b1[None])
        pre = sc_gather(a, b, src, dst)
        m = _tc_edge(pre, p['e2'][0].T, p['e2'][1][None])
        return sc_scatter(m, dst, zeros_tbl)

    g0, g1 = layer(x, params['l0'])
    wa, ba, wb, bb = _node_weights(params['l0'])
    h1 = _tc_node_ln(g0, g1, x, wa, ba, wb, bb,
                     params['ln0']['g'][None], params['ln0']['b'][None])
    g0, g1 = layer(h1, params['l1'])
    wa, ba, wb, bb = _node_weights(params['l1'])
    dummy = jnp.zeros((1, D), jnp.float32)
    return _tc_node_plain(g0, g1, h1, wa, ba, wb, bb, dummy, dummy)


# speedup vs baseline: 5.5303x; 1.1720x over previous
import functools

import jax
import jax.numpy as jnp
from jax import lax
from jax.experimental import pallas as pl
from jax.experimental.pallas import tpu as pltpu
from jax.experimental.pallas import tpu_sc as plsc

N = 10000
E = 320000
D = 128

NC = 2
NS = 16
NW = NC * NS
EPW = E // NW
CH = 128
NCH = EPW // CH
NJ = NCH // 2
TAIL = EPW - NCH * CH
ZCH = 400
NZC = N // ZCH


@functools.lru_cache(maxsize=1)
def _sc_kernels():
    mesh = plsc.VectorSubcoreMesh(core_axis_name="c", subcore_axis_name="s",
                                  num_cores=NC, num_subcores=NS)

    def add_into(wo, ra, rb, n_rows):
        def body(r, carry):
            for c in range(D // 16):
                sl = pl.ds(c * 16, 16)
                wo[r, sl] = ra[r, sl] + rb[r, sl]
            return carry

        lax.fori_loop(0, n_rows, body, 0)

    @functools.partial(
        pl.kernel,
        mesh=mesh,
        out_type=jax.ShapeDtypeStruct((E, D), jnp.float32),
        scratch_types=[
            pltpu.VMEM((EPW,), jnp.int32),
            pltpu.VMEM((EPW,), jnp.int32),
            pltpu.VMEM((CH, D), jnp.float32),
            pltpu.VMEM((CH, D), jnp.float32),
            pltpu.VMEM((CH, D), jnp.float32),
            pltpu.VMEM((CH, D), jnp.float32),
            pltpu.VMEM((CH, D), jnp.float32),
            pltpu.VMEM((CH, D), jnp.float32),
            pltpu.SemaphoreType.DMA,
            pltpu.SemaphoreType.DMA,
            pltpu.SemaphoreType.DMA,
            pltpu.SemaphoreType.DMA,
        ],
    )
    def sc_gather(a_hbm, b_hbm, src_hbm, dst_hbm, out_hbm,
                  si, di, ra0, rb0, ra1, rb1, wo0, wo1,
                  gs0, gs1, ws0, ws1):
        wid = lax.axis_index("s") * NC + lax.axis_index("c")
        base = wid * EPW
        pltpu.sync_copy(src_hbm.at[pl.ds(base, EPW)], si)
        pltpu.sync_copy(dst_hbm.at[pl.ds(base, EPW)], di)

        def gather(c, ra, rb, gs):
            l = c * CH
            pltpu.async_copy(a_hbm.at[si.at[pl.ds(l, CH)]], ra, gs)
            pltpu.async_copy(b_hbm.at[di.at[pl.ds(l, CH)]], rb, gs)

        def drain(buf_a, buf_b, sem):
            pltpu.make_async_copy(a_hbm.at[pl.ds(0, CH)], buf_a, sem).wait()
            pltpu.make_async_copy(a_hbm.at[pl.ds(0, CH)], buf_b, sem).wait()

        def drain_w(buf, sem):
            pltpu.make_async_copy(buf, out_hbm.at[pl.ds(base, CH)], sem).wait()

        gather(0, ra0, rb0, gs0)
        gather(1, ra1, rb1, gs1)

        def pair(j, carry):
            c0 = 2 * j
            drain(ra0, rb0, gs0)

            @pl.when(j > 0)
            def _():
                drain_w(wo0, ws0)

            add_into(wo0, ra0, rb0, CH)

            @pl.when(j < NJ - 1)
            def _():
                gather(c0 + 2, ra0, rb0, gs0)

            pltpu.async_copy(wo0, out_hbm.at[pl.ds(base + c0 * CH, CH)], ws0)

            drain(ra1, rb1, gs1)

            @pl.when(j > 0)
            def _():
                drain_w(wo1, ws1)

            add_into(wo1, ra1, rb1, CH)

            @pl.when(j < NJ - 1)
            def _():
                gather(c0 + 3, ra1, rb1, gs1)

            pltpu.async_copy(wo1, out_hbm.at[pl.ds(base + (c0 + 1) * CH, CH)],
                             ws1)
            return carry

        lax.fori_loop(0, NJ, pair, 0)
        drain_w(wo0, ws0)
        drain_w(wo1, ws1)

        lt = EPW - CH
        ca = pltpu.async_copy(a_hbm.at[si.at[pl.ds(lt, CH)]], ra0, gs0)
        cb = pltpu.async_copy(b_hbm.at[di.at[pl.ds(lt, CH)]], rb0, gs0)
        ca.wait()
        cb.wait()
        add_into(wo0, ra0, rb0, CH)
        pltpu.sync_copy(wo0, out_hbm.at[pl.ds(base + lt, CH)])

    @functools.partial(
        pl.kernel,
        mesh=mesh,
        out_type=(jax.ShapeDtypeStruct((N, D), jnp.float32),
                  jax.ShapeDtypeStruct((N, D), jnp.float32)),
        scratch_types=[
            pltpu.VMEM((EPW,), jnp.int32),
            pltpu.VMEM((CH, D), jnp.float32),
            pltpu.VMEM((CH, D), jnp.float32),
            pltpu.VMEM((TAIL, D), jnp.float32),
            pltpu.VMEM_SHARED((N, D), jnp.float32),
            pltpu.SemaphoreType.DMA,
            pltpu.SemaphoreType.DMA,
        ],
    )
    def sc_scatter(m_hbm, dst_hbm, zeros_hbm, out0_hbm, out1_hbm,
                   di, r0, r1, rt, acc, s0, s1):
        cid = lax.axis_index("c")
        sid = lax.axis_index("s")
        wid = sid * NC + cid
        base = wid * EPW

        def chunk_copy(ci, src_ref, dst_ref):
            sl = pl.ds(ci * ZCH, ZCH)
            pltpu.sync_copy(src_ref.at[sl], dst_ref.at[sl])

        chunk_copy(sid, zeros_hbm, acc)

        @pl.when(sid + NS < NZC)
        def _():
            chunk_copy(sid + NS, zeros_hbm, acc)

        pltpu.sync_copy(dst_hbm.at[pl.ds(base, EPW)], di)
        plsc.subcore_barrier()

        def pair(j, carry):
            l0 = (2 * j) * CH
            l1 = l0 + CH
            c0 = pltpu.async_copy(m_hbm.at[pl.ds(base + l0, CH)], r0, s0)
            c1 = pltpu.async_copy(m_hbm.at[pl.ds(base + l1, CH)], r1, s1)
            c0.wait()
            pltpu.sync_copy(r0, acc.at[di.at[pl.ds(l0, CH)]], add=True)
            c1.wait()
            pltpu.sync_copy(r1, acc.at[di.at[pl.ds(l1, CH)]], add=True)
            return carry

        lax.fori_loop(0, NJ, pair, 0)

        lt = NCH * CH
        pltpu.sync_copy(m_hbm.at[pl.ds(base + lt, TAIL)], rt)
        pltpu.sync_copy(rt, acc.at[di.at[pl.ds(lt, TAIL)]], add=True)

        plsc.subcore_barrier()

        @pl.when(cid == 0)
        def _():
            chunk_copy(sid, acc, out0_hbm)

            @pl.when(sid + NS < NZC)
            def _():
                chunk_copy(sid + NS, acc, out0_hbm)

        @pl.when(cid == 1)
        def _():
            chunk_copy(sid, acc, out1_hbm)

            @pl.when(sid + NS < NZC)
            def _():
                chunk_copy(sid + NS, acc, out1_hbm)

    return sc_gather, sc_scatter


BN = 2000
BE = 4000


def _pre_body(h_ref, wst_ref, wdt_ref, b1_ref, a_ref, b_ref):
    h = h_ref[...]
    a_ref[...] = jnp.dot(h, wst_ref[...], preferred_element_type=jnp.float32)
    b_ref[...] = (jnp.dot(h, wdt_ref[...], preferred_element_type=jnp.float32)
                  + b1_ref[...])


_tc_pre = pl.pallas_call(
    _pre_body,
    grid=(N // BN,),
    in_specs=[
        pl.BlockSpec((BN, D), lambda i: (i, 0)),
        pl.BlockSpec((D, D), lambda i: (0, 0)),
        pl.BlockSpec((D, D), lambda i: (0, 0)),
        pl.BlockSpec((1, D), lambda i: (0, 0)),
    ],
    out_specs=(pl.BlockSpec((BN, D), lambda i: (i, 0)),
               pl.BlockSpec((BN, D), lambda i: (i, 0))),
    out_shape=(jax.ShapeDtypeStruct((N, D), jnp.float32),
               jax.ShapeDtypeStruct((N, D), jnp.float32)),
)


def _edge_body(pre_ref, w2t_ref, b2_ref, m_ref):
    z = jnp.maximum(pre_ref[...], 0.0)
    m_ref[...] = jnp.tanh(
        jnp.dot(z, w2t_ref[...], preferred_element_type=jnp.float32)
        + b2_ref[...])


_tc_edge = pl.pallas_call(
    _edge_body,
    grid=(E // BE,),
    in_specs=[
        pl.BlockSpec((BE, D), lambda i: (i, 0)),
        pl.BlockSpec((D, D), lambda i: (0, 0)),
        pl.BlockSpec((1, D), lambda i: (0, 0)),
    ],
    out_specs=pl.BlockSpec((BE, D), lambda i: (i, 0)),
    out_shape=jax.ShapeDtypeStruct((E, D), jnp.float32),
)


def _node_body(g0_ref, g1_ref, h_ref, wa_ref, ba_ref, wb_ref, bb_ref,
               g_ref, beta_ref, out_ref, *, ln):
    agg = g0_ref[...] + g1_ref[...]
    cat = jnp.concatenate([agg, h_ref[...]], axis=1)
    t = jnp.maximum(
        jnp.dot(cat, wa_ref[...], preferred_element_type=jnp.float32)
        + ba_ref[...], 0.0)
    y = jnp.tanh(
        jnp.dot(t, wb_ref[...], preferred_element_type=jnp.float32)
        + bb_ref[...])
    if ln:
        r = jnp.maximum(y, 0.0)
        mu = jnp.mean(r, axis=1, keepdims=True)
        var = jnp.mean((r - mu) ** 2, axis=1, keepdims=True)
        y = (r - mu) * jax.lax.rsqrt(var + 1e-5) * g_ref[...] + beta_ref[...]
    out_ref[...] = y


def _make_tc_node(ln):
    return pl.pallas_call(
        functools.partial(_node_body, ln=ln),
        grid=(N // BN,),
        in_specs=[
            pl.BlockSpec((BN, D), lambda i: (i, 0)),
            pl.BlockSpec((BN, D), lambda i: (i, 0)),
            pl.BlockSpec((BN, D), lambda i: (i, 0)),
            pl.BlockSpec((2 * D, 2 * D), lambda i: (0, 0)),
            pl.BlockSpec((1, 2 * D), lambda i: (0, 0)),
            pl.BlockSpec((2 * D, D), lambda i: (0, 0)),
            pl.BlockSpec((1, D), lambda i: (0, 0)),
            pl.BlockSpec((1, D), lambda i: (0, 0)),
            pl.BlockSpec((1, D), lambda i: (0, 0)),
        ],
        out_specs=pl.BlockSpec((BN, D), lambda i: (i, 0)),
        out_shape=jax.ShapeDtypeStruct((N, D), jnp.float32),
    )


_tc_node_ln = _make_tc_node(True)
_tc_node_plain = _make_tc_node(False)


def _node_weights(p):
    w1a, b1a = p['n1a']
    w2a, b2a = p['n2a']
    w1b, b1b = p['n1b']
    w2b, b2b = p['n2b']
    wa = jnp.zeros((2 * D, 2 * D), jnp.float32)
    wa = wa.at[:D, :D].set(w1a.T).at[D:, D:].set(w2a.T)
    ba = jnp.concatenate([b1a, b2a])[None]
    wb = jnp.zeros((2 * D, D), jnp.float32)
    wb = wb.at[:D, :D // 2].set(w1b.T).at[D:, D // 2:].set(w2b.T)
    bb = jnp.concatenate([b1b, b2b])[None]
    return wa, ba, wb, bb


def kernel(x, edge_index, params):
    sc_gather, sc_scatter = _sc_kernels()
    src = edge_index[0]
    dst = edge_index[1]
    zeros_tbl = jnp.zeros((N, D), jnp.float32)

    def layer(h, p):
        w1, b1 = p['e1']
        a, b = _tc_pre(h, w1[:, :D].T, w1[:, D:].T, b1[None])
        pre = sc_gather(a, b, src, dst)
        m = _tc_edge(pre, p['e2'][0].T, p['e2'][1][None])
        return sc_scatter(m, dst, zeros_tbl)

    g0, g1 = layer(x, params['l0'])
    wa, ba, wb, bb = _node_weights(params['l0'])
    h1 = _tc_node_ln(g0, g1, x, wa, ba, wb, bb,
                     params['ln0']['g'][None], params['ln0']['b'][None])
    g0, g1 = layer(h1, params['l1'])
    wa, ba, wb, bb = _node_weights(params['l1'])
    dummy = jnp.zeros((1, D), jnp.float32)
    return _tc_node_plain(g0, g1, h1, wa, ba, wb, bb, dummy, dummy)


# speedup vs baseline: 5.7896x; 1.0469x over previous
import functools

import jax
import jax.numpy as jnp
from jax import lax
from jax.experimental import pallas as pl
from jax.experimental.pallas import tpu as pltpu
from jax.experimental.pallas import tpu_sc as plsc

N = 10000
E = 320000
D = 128

NC = 2
NS = 16
NW = NC * NS
EH = E // 2
ZCH = 400
NZC = N // ZCH


@functools.lru_cache(maxsize=4)
def _sc_kernels(ne):
    epw = ne // NW
    ch = 104
    nch = epw // ch
    nj = nch // 2
    assert nch == 2 * nj and epw >= ch
    tail = epw - nch * ch
    mesh = plsc.VectorSubcoreMesh(core_axis_name="c", subcore_axis_name="s",
                                  num_cores=NC, num_subcores=NS)

    def add_into(wo, ra, rb, n_rows):
        def body(r, carry):
            for c in range(D // 16):
                sl = pl.ds(c * 16, 16)
                wo[r, sl] = ra[r, sl] + rb[r, sl]
            return carry

        lax.fori_loop(0, n_rows, body, 0)

    @functools.partial(
        pl.kernel,
        mesh=mesh,
        out_type=jax.ShapeDtypeStruct((ne, D), jnp.float32),
        scratch_types=[
            pltpu.VMEM((epw,), jnp.int32),
            pltpu.VMEM((epw,), jnp.int32),
            pltpu.VMEM((ch, D), jnp.float32),
            pltpu.VMEM((ch, D), jnp.float32),
            pltpu.VMEM((ch, D), jnp.float32),
            pltpu.VMEM((ch, D), jnp.float32),
            pltpu.VMEM((ch, D), jnp.float32),
            pltpu.VMEM((ch, D), jnp.float32),
            pltpu.SemaphoreType.DMA,
            pltpu.SemaphoreType.DMA,
            pltpu.SemaphoreType.DMA,
            pltpu.SemaphoreType.DMA,
        ],
    )
    def sc_gather(a_hbm, b_hbm, src_hbm, dst_hbm, out_hbm,
                  si, di, ra0, rb0, ra1, rb1, wo0, wo1,
                  gs0, gs1, ws0, ws1):
        wid = lax.axis_index("s") * NC + lax.axis_index("c")
        base = wid * epw
        pltpu.sync_copy(src_hbm.at[pl.ds(base, epw)], si)
        pltpu.sync_copy(dst_hbm.at[pl.ds(base, epw)], di)

        def gather(c, ra, rb, gs):
            l = c * ch
            pltpu.async_copy(a_hbm.at[si.at[pl.ds(l, ch)]], ra, gs)
            pltpu.async_copy(b_hbm.at[di.at[pl.ds(l, ch)]], rb, gs)

        def drain(buf_a, buf_b, sem):
            pltpu.make_async_copy(a_hbm.at[pl.ds(0, ch)], buf_a, sem).wait()
            pltpu.make_async_copy(a_hbm.at[pl.ds(0, ch)], buf_b, sem).wait()

        def drain_w(buf, sem):
            pltpu.make_async_copy(buf, out_hbm.at[pl.ds(base, ch)], sem).wait()

        gather(0, ra0, rb0, gs0)
        gather(1, ra1, rb1, gs1)

        def pair(j, carry):
            c0 = 2 * j
            drain(ra0, rb0, gs0)

            @pl.when(j > 0)
            def _():
                drain_w(wo0, ws0)

            add_into(wo0, ra0, rb0, ch)

            @pl.when(j < nj - 1)
            def _():
                gather(c0 + 2, ra0, rb0, gs0)

            pltpu.async_copy(wo0, out_hbm.at[pl.ds(base + c0 * ch, ch)], ws0)

            drain(ra1, rb1, gs1)

            @pl.when(j > 0)
            def _():
                drain_w(wo1, ws1)

            add_into(wo1, ra1, rb1, ch)

            @pl.when(j < nj - 1)
            def _():
                gather(c0 + 3, ra1, rb1, gs1)

            pltpu.async_copy(wo1, out_hbm.at[pl.ds(base + (c0 + 1) * ch, ch)],
                             ws1)
            return carry

        lax.fori_loop(0, nj, pair, 0)
        drain_w(wo0, ws0)
        drain_w(wo1, ws1)

        lt = epw - ch
        ca = pltpu.async_copy(a_hbm.at[si.at[pl.ds(lt, ch)]], ra0, gs0)
        cb = pltpu.async_copy(b_hbm.at[di.at[pl.ds(lt, ch)]], rb0, gs0)
        ca.wait()
        cb.wait()
        add_into(wo0, ra0, rb0, ch)
        pltpu.sync_copy(wo0, out_hbm.at[pl.ds(base + lt, ch)])

    @functools.partial(
        pl.kernel,
        mesh=mesh,
        out_type=(jax.ShapeDtypeStruct((N, D), jnp.float32),
                  jax.ShapeDtypeStruct((N, D), jnp.float32)),
        scratch_types=[
            pltpu.VMEM((epw,), jnp.int32),
            pltpu.VMEM((ch, D), jnp.float32),
            pltpu.VMEM((ch, D), jnp.float32),
            pltpu.VMEM((max(tail, 1), D), jnp.float32),
            pltpu.VMEM_SHARED((N, D), jnp.float32),
            pltpu.SemaphoreType.DMA,
            pltpu.SemaphoreType.DMA,
        ],
    )
    def sc_scatter(m_hbm, dst_hbm, init0_hbm, init1_hbm, out0_hbm, out1_hbm,
                   di, r0, r1, rt, acc, s0, s1):
        cid = lax.axis_index("c")
        sid = lax.axis_index("s")
        wid = sid * NC + cid
        base = wid * epw

        def chunk_copy(ci, src_ref, dst_ref):
            sl = pl.ds(ci * ZCH, ZCH)
            pltpu.sync_copy(src_ref.at[sl], dst_ref.at[sl])

        def seed(init_hbm):
            chunk_copy(sid, init_hbm, acc)

            @pl.when(sid + NS < NZC)
            def _():
                chunk_copy(sid + NS, init_hbm, acc)

        @pl.when(cid == 0)
        def _():
            seed(init0_hbm)

        @pl.when(cid == 1)
        def _():
            seed(init1_hbm)

        pltpu.sync_copy(dst_hbm.at[pl.ds(base, epw)], di)
        plsc.subcore_barrier()

        def pair(j, carry):
            l0 = (2 * j) * ch
            l1 = l0 + ch
            c0 = pltpu.async_copy(m_hbm.at[pl.ds(base + l0, ch)], r0, s0)
            c1 = pltpu.async_copy(m_hbm.at[pl.ds(base + l1, ch)], r1, s1)
            c0.wait()
            pltpu.sync_copy(r0, acc.at[di.at[pl.ds(l0, ch)]], add=True)
            c1.wait()
            pltpu.sync_copy(r1, acc.at[di.at[pl.ds(l1, ch)]], add=True)
            return carry

        lax.fori_loop(0, nj, pair, 0)

        if tail:
            lt = nch * ch
            pltpu.sync_copy(m_hbm.at[pl.ds(base + lt, tail)], rt)
            pltpu.sync_copy(rt, acc.at[di.at[pl.ds(lt, tail)]], add=True)

        plsc.subcore_barrier()

        @pl.when(cid == 0)
        def _():
            chunk_copy(sid, acc, out0_hbm)

            @pl.when(sid + NS < NZC)
            def _():
                chunk_copy(sid + NS, acc, out0_hbm)

        @pl.when(cid == 1)
        def _():
            chunk_copy(sid, acc, out1_hbm)

            @pl.when(sid + NS < NZC)
            def _():
                chunk_copy(sid + NS, acc, out1_hbm)

    return sc_gather, sc_scatter


BN = 2000
BE = 4000


def _pre_body(h_ref, wst_ref, wdt_ref, b1_ref, a_ref, b_ref):
    h = h_ref[...]
    a_ref[...] = jnp.dot(h, wst_ref[...], preferred_element_type=jnp.float32)
    b_ref[...] = (jnp.dot(h, wdt_ref[...], preferred_element_type=jnp.float32)
                  + b1_ref[...])


_tc_pre = pl.pallas_call(
    _pre_body,
    grid=(N // BN,),
    in_specs=[
        pl.BlockSpec((BN, D), lambda i: (i, 0)),
        pl.BlockSpec((D, D), lambda i: (0, 0)),
        pl.BlockSpec((D, D), lambda i: (0, 0)),
        pl.BlockSpec((1, D), lambda i: (0, 0)),
    ],
    out_specs=(pl.BlockSpec((BN, D), lambda i: (i, 0)),
               pl.BlockSpec((BN, D), lambda i: (i, 0))),
    out_shape=(jax.ShapeDtypeStruct((N, D), jnp.float32),
               jax.ShapeDtypeStruct((N, D), jnp.float32)),
)


def _edge_body(pre_ref, w2t_ref, b2_ref, m_ref):
    z = jnp.maximum(pre_ref[...], 0.0)
    m_ref[...] = jnp.tanh(
        jnp.dot(z, w2t_ref[...], preferred_element_type=jnp.float32)
        + b2_ref[...])


_tc_edge = pl.pallas_call(
    _edge_body,
    grid=(EH // BE,),
    in_specs=[
        pl.BlockSpec((BE, D), lambda i: (i, 0)),
        pl.BlockSpec((D, D), lambda i: (0, 0)),
        pl.BlockSpec((1, D), lambda i: (0, 0)),
    ],
    out_specs=pl.BlockSpec((BE, D), lambda i: (i, 0)),
    out_shape=jax.ShapeDtypeStruct((EH, D), jnp.float32),
)


def _node_body(g0_ref, g1_ref, h_ref, wa_ref, ba_ref, wb_ref, bb_ref,
               g_ref, beta_ref, out_ref, *, ln):
    agg = g0_ref[...] + g1_ref[...]
    cat = jnp.concatenate([agg, h_ref[...]], axis=1)
    t = jnp.maximum(
        jnp.dot(cat, wa_ref[...], preferred_element_type=jnp.float32)
        + ba_ref[...], 0.0)
    y = jnp.tanh(
        jnp.dot(t, wb_ref[...], preferred_element_type=jnp.float32)
        + bb_ref[...])
    if ln:
        r = jnp.maximum(y, 0.0)
        mu = jnp.mean(r, axis=1, keepdims=True)
        var = jnp.mean((r - mu) ** 2, axis=1, keepdims=True)
        y = (r - mu) * jax.lax.rsqrt(var + 1e-5) * g_ref[...] + beta_ref[...]
    out_ref[...] = y


def _make_tc_node(ln):
    return pl.pallas_call(
        functools.partial(_node_body, ln=ln),
        grid=(N // BN,),
        in_specs=[
            pl.BlockSpec((BN, D), lambda i: (i, 0)),
            pl.BlockSpec((BN, D), lambda i: (i, 0)),
            pl.BlockSpec((BN, D), lambda i: (i, 0)),
            pl.BlockSpec((2 * D, 2 * D), lambda i: (0, 0)),
            pl.BlockSpec((1, 2 * D), lambda i: (0, 0)),
            pl.BlockSpec((2 * D, D), lambda i: (0, 0)),
            pl.BlockSpec((1, D), lambda i: (0, 0)),
            pl.BlockSpec((1, D), lambda i: (0, 0)),
            pl.BlockSpec((1, D), lambda i: (0, 0)),
        ],
        out_specs=pl.BlockSpec((BN, D), lambda i: (i, 0)),
        out_shape=jax.ShapeDtypeStruct((N, D), jnp.float32),
    )


_tc_node_ln = _make_tc_node(True)
_tc_node_plain = _make_tc_node(False)


def _node_weights(p):
    w1a, b1a = p['n1a']
    w2a, b2a = p['n2a']
    w1b, b1b = p['n1b']
    w2b, b2b = p['n2b']
    wa = jnp.zeros((2 * D, 2 * D), jnp.float32)
    wa = wa.at[:D, :D].set(w1a.T).at[D:, D:].set(w2a.T)
    ba = jnp.concatenate([b1a, b2a])[None]
    wb = jnp.zeros((2 * D, D), jnp.float32)
    wb = wb.at[:D, :D // 2].set(w1b.T).at[D:, D // 2:].set(w2b.T)
    bb = jnp.concatenate([b1b, b2b])[None]
    return wa, ba, wb, bb


def kernel(x, edge_index, params):
    sc_gather, sc_scatter = _sc_kernels(EH)
    src = edge_index[0]
    dst = edge_index[1]
    halves = [(src[:EH], dst[:EH]), (src[EH:], dst[EH:])]
    zeros_tbl = jnp.zeros((N, D), jnp.float32)

    def layer(h, p):
        w1, b1 = p['e1']
        a, b = _tc_pre(h, w1[:, :D].T, w1[:, D:].T, b1[None])
        w2t, b2 = p['e2'][0].T, p['e2'][1][None]
        pre0 = sc_gather(a, b, *halves[0])
        pre1 = sc_gather(a, b, *halves[1])
        m0 = _tc_edge(pre0, w2t, b2)
        m1 = _tc_edge(pre1, w2t, b2)
        p0, p1 = sc_scatter(m0, halves[0][1], zeros_tbl, zeros_tbl)
        return sc_scatter(m1, halves[1][1], p0, p1)

    g0, g1 = layer(x, params['l0'])
    wa, ba, wb, bb = _node_weights(params['l0'])
    h1 = _tc_node_ln(g0, g1, x, wa, ba, wb, bb,
                     params['ln0']['g'][None], params['ln0']['b'][None])
    g0, g1 = layer(h1, params['l1'])
    wa, ba, wb, bb = _node_weights(params['l1'])
    dummy = jnp.zeros((1, D), jnp.float32)
    return _tc_node_plain(g0, g1, h1, wa, ba, wb, bb, dummy, dummy)
